# baseline (speedup 1.0000x reference)
import functools

import jax
import jax.numpy as jnp
from jax import lax
from jax.experimental import pallas as pl
from jax.experimental.pallas import tpu as pltpu
from jax.experimental.pallas import tpu_sc as plsc

B, T, V = 1024, 50, 1000
N = B * T
NC, NS, L = 2, 16, 16
NW = NC * NS
PER = N // NW
CHUNK = 80
NCHUNK = PER // CHUNK
WAVE = NCHUNK // 2
HALF = PER // 2
BLK = 128
CPW = B // NW

_mesh = plsc.VectorSubcoreMesh(core_axis_name="c", subcore_axis_name="s")


@functools.partial(
    pl.kernel,
    out_type=jax.ShapeDtypeStruct((NW, L), jnp.float32),
    mesh=_mesh,
    scratch_types=[
        pltpu.VMEM((PER,), jnp.int32),
        pltpu.VMEM((PER,), jnp.float32),
        pltpu.VMEM((T, BLK), jnp.float32),
        pltpu.VMEM((L,), jnp.float32),
        pltpu.SemaphoreType.DMA,
        pltpu.SemaphoreType.DMA,
    ],
)
def _sc_gather(flat_hbm, rows_hbm, mask_hbm, prod_out,
               idx_v, vals_v, mask_v, acc_v, sem_a, sem_b):
    wid = lax.axis_index("s") * NC + lax.axis_index("c")
    base = wid * PER
    col0 = pl.multiple_of((wid % 4) * CPW, 8)

    pltpu.sync_copy(rows_hbm.at[pl.ds(base, PER)], idx_v)
    bcol = pl.multiple_of((wid // 4) * BLK, BLK)
    pltpu.sync_copy(mask_hbm.at[:, pl.ds(bcol, BLK)], mask_v)

    def _chunk_copy(c, sem):
        sl = pl.ds(pl.multiple_of(c * CHUNK, 8), CHUNK)
        return pltpu.make_async_copy(
            flat_hbm.at[idx_v.at[sl]], vals_v.at[sl], sem)

    @pl.loop(0, WAVE)
    def _(c):
        _chunk_copy(c, sem_a).start()

    @pl.loop(WAVE, NCHUNK)
    def _(c):
        _chunk_copy(c, sem_b).start()

    @pl.loop(0, WAVE)
    def _(c):
        _chunk_copy(c, sem_a).wait()

    def _acc_wave(lo, hi, init):
        def body(j, acc):
            t = j // 2
            s = lax.rem(j, 2)
            c = pl.multiple_of(col0 + s * L, 8)
            m = mask_v[t, pl.ds(c, L)]
            v = vals_v[pl.ds(pl.multiple_of(j * L, 8), L)]
            return acc + v * m
        return lax.fori_loop(lo, hi, body, init)

    acc = _acc_wave(0, HALF // L, jnp.zeros((L,), jnp.float32))

    @pl.loop(WAVE, NCHUNK)
    def _(c):
        _chunk_copy(c, sem_b).wait()

    acc = _acc_wave(HALF // L, PER // L, acc)

    acc_v[...] = acc
    pltpu.sync_copy(acc_v, prod_out.at[wid])


def _finish_body(p_ref, m_ref, o_ref):
    s = -jnp.sum(p_ref[...]) / jnp.sum(m_ref[...])
    o_ref[...] = jnp.full((1, 1), s, jnp.float32)


_finish = pl.pallas_call(
    _finish_body,
    out_shape=jax.ShapeDtypeStruct((1, 1), jnp.float32),
)


def kernel(input, target, mask):
    x1 = (input.transpose(1, 2, 0)
          .reshape(T, V // 8, 8, B // 128, 128)
          .transpose(0, 1, 3, 2, 4)
          .reshape(-1))
    tT = target.T.astype(jnp.int32)
    t_col = jnp.arange(T, dtype=jnp.int32)[:, None]
    b_row = jnp.arange(B, dtype=jnp.int32)[None, :]
    rows = ((t_col * (V // 8) + tT // 8) * 64 + (b_row // 128) * 8
            + tT % 8) * 128 + b_row % 128
    rows_g = rows.reshape(T, NW, 2, L).transpose(1, 0, 2, 3).reshape(-1)
    mT = mask.T
    prod_p = _sc_gather(x1, rows_g, mT)
    return _finish(prod_p, mT)[0, 0]


# speedup vs baseline: 1.1444x; 1.1444x over previous
import functools

import jax
import jax.numpy as jnp
from jax import lax
from jax.experimental import pallas as pl
from jax.experimental.pallas import tpu as pltpu
from jax.experimental.pallas import tpu_sc as plsc

B, T, V = 1024, 50, 1000
N = B * T
NC, NS, L = 2, 16, 16
NW = NC * NS
PER = N // NW
CHUNK = 80
NCHUNK = PER // CHUNK
WAVE = NCHUNK // 2
TH = T // 2
BLK = 128
CPW = B // NW

_mesh = plsc.VectorSubcoreMesh(core_axis_name="c", subcore_axis_name="s")


@functools.partial(
    pl.kernel,
    out_type=jax.ShapeDtypeStruct((NW, L), jnp.float32),
    mesh=_mesh,
    scratch_types=[
        pltpu.VMEM((T, BLK), jnp.int32),
        pltpu.VMEM((T, BLK), jnp.float32),
        pltpu.VMEM((PER,), jnp.int32),
        pltpu.VMEM((PER,), jnp.float32),
        pltpu.VMEM((L,), jnp.float32),
        pltpu.SemaphoreType.DMA,
        pltpu.SemaphoreType.DMA,
    ],
)
def _sc_gather(flat_hbm, tgt_hbm, mask_hbm, prod_out,
               tgt_v, mask_v, idx_v, vals_v, acc_v, sem_a, sem_b):
    wid = lax.axis_index("s") * NC + lax.axis_index("c")
    blk = wid // 4
    col0 = pl.multiple_of((wid % 4) * CPW, 8)

    bcol = pl.multiple_of(blk * BLK, BLK)
    pltpu.sync_copy(tgt_hbm.at[:, pl.ds(bcol, BLK)], tgt_v)
    pltpu.sync_copy(mask_hbm.at[:, pl.ds(bcol, BLK)], mask_v)

    lane = lax.iota(jnp.int32, L)

    def _compute_idx(tlo, thi):
        @pl.loop(tlo, thi)
        def _(j):
            for s in range(2):
                c = pl.multiple_of(col0 + s * L, 8)
                t16 = tgt_v[j, pl.ds(c, L)]
                idx16 = (j * (V * B) + (t16 >> 3) * 8192 + blk * 1024
                         + (t16 & 7) * BLK + c + lane)
                idx_v[pl.ds(pl.multiple_of(j * 2 * L, 8) + s * L, L)] = idx16

    def _chunk_copy(c, sem):
        sl = pl.ds(pl.multiple_of(c * CHUNK, 8), CHUNK)
        return pltpu.make_async_copy(
            flat_hbm.at[idx_v.at[sl]], vals_v.at[sl], sem)

    def _acc_wave(lo, hi, init):
        def body(j, acc):
            t = j // 2
            s = lax.rem(j, 2)
            c = pl.multiple_of(col0 + s * L, 8)
            m = mask_v[t, pl.ds(c, L)]
            v = vals_v[pl.ds(pl.multiple_of(j * L, 8), L)]
            return acc + v * m
        return lax.fori_loop(lo, hi, body, init)

    _compute_idx(0, TH)

    @pl.loop(0, WAVE)
    def _(c):
        _chunk_copy(c, sem_a).start()

    _compute_idx(TH, T)

    @pl.loop(WAVE, NCHUNK)
    def _(c):
        _chunk_copy(c, sem_b).start()

    @pl.loop(0, WAVE)
    def _(c):
        _chunk_copy(c, sem_a).wait()

    acc = _acc_wave(0, T, jnp.zeros((L,), jnp.float32))

    @pl.loop(WAVE, NCHUNK)
    def _(c):
        _chunk_copy(c, sem_b).wait()

    acc = _acc_wave(T, 2 * T, acc)

    acc_v[...] = acc
    pltpu.sync_copy(acc_v, prod_out.at[wid])


def _finish_body(p_ref, m_ref, o_ref):
    s = -jnp.sum(p_ref[...]) / jnp.sum(m_ref[...])
    o_ref[...] = jnp.full((1, 1), s, jnp.float32)


_finish = pl.pallas_call(
    _finish_body,
    out_shape=jax.ShapeDtypeStruct((1, 1), jnp.float32),
)


def kernel(input, target, mask):
    x1 = (input.transpose(1, 2, 0)
          .reshape(T, V // 8, 8, B // 128, 128)
          .transpose(0, 1, 3, 2, 4)
          .reshape(-1))
    mT = mask.T
    prod_p = _sc_gather(x1, target.T, mT)
    return _finish(prod_p, mT)[0, 0]
